# Initial kernel scaffold; baseline (speedup 1.0000x reference)
import jax
import jax.numpy as jnp
from jax.experimental import pallas as pl


def kernel(bboxes_txtytwth, anchors, scores, image_shape):
    raise NotImplementedError("write your pallas kernel here")


# speedup vs baseline: 11.3659x; 11.3659x over previous
import functools

import jax
import jax.numpy as jnp
from jax import lax
from jax.experimental import pallas as pl
from jax.experimental.pallas import tpu as pltpu

_NUM_ANCHORS = 9
_NUM_PRE_NMS = 6000
_NUM_POST_NMS = 300
_IOU_THR = 0.7
_MEANS = jnp.array([0.0, 0.0, 0.0, 0.0], dtype=jnp.float32)
_STDS = jnp.array([1.0, 1.0, 1.0, 1.0], dtype=jnp.float32)

_N = 22500
_NP = 22528
_NROWS = 176
_KP = 6016
_NBLK = 47
_SCAP = 512

_F32 = jnp.float32

_dg = functools.partial(
    lax.dot_general,
    precision=lax.Precision.HIGHEST,
    preferred_element_type=jnp.float32,
)


def _nms_body(fg_row_ref, fg_col_ref, boxes_ref, out_ref,
              rank_ref, sorted_ref, surv_ref, sarea_ref):
    eye = (lax.broadcasted_iota(jnp.int32, (128, 128), 0)
           == lax.broadcasted_iota(jnp.int32, (128, 128), 1)).astype(_F32)
    ltmat = (lax.broadcasted_iota(jnp.int32, (128, 128), 0)
             < lax.broadcasted_iota(jnp.int32, (128, 128), 1)).astype(_F32)
    lane_i = lax.broadcasted_iota(jnp.int32, (1, 128), 1)
    sub_i = lax.broadcasted_iota(jnp.int32, (128, 1), 0)
    lane_f = lane_i.astype(_F32)

    def tcol(row):
        return _dg(eye, row, (((1,), (1,)), ((), ())))

    def trow(col):
        return _dg(col, eye, (((0,), (0,)), ((), ())))

    def rank_row(r, carry):
        fgi = fg_row_ref[pl.ds(r, 1), :]

        def lo(jc, acc):
            fgj = fg_col_ref[pl.ds(jc * 128, 128), :]
            return acc + (fgj >= fgi).astype(_F32)

        def hi(jc, acc):
            fgj = fg_col_ref[pl.ds(jc * 128, 128), :]
            return acc + (fgj > fgi).astype(_F32)

        acc = jnp.zeros((128, 128), _F32)
        acc = lax.fori_loop(0, r, lo, acc)
        acc = lax.fori_loop(r + 1, _NROWS, hi, acc)
        fgd = fg_col_ref[pl.ds(r * 128, 128), :]
        acc = acc + ((fgd > fgi)
                     | ((fgd == fgi) & (sub_i < lane_i))).astype(_F32)
        rrow = jnp.sum(acc, axis=0, keepdims=True)
        rank_ref[pl.ds(r * 128, 128), :] = tcol(rrow)
        return carry

    lax.fori_loop(0, _NROWS, rank_row, 0)

    sorted_ref[...] = jnp.zeros((_KP, 4), _F32)

    def scat_i(ic, carry):
        rcol = rank_ref[pl.ds(ic * 128, 128), :]
        pay = boxes_ref[pl.ds(ic * 128, 128), :]

        def scat_s(sc, c2):
            base = (sc * 128).astype(_F32)
            oh = jnp.where(rcol == base + lane_f, 1.0, 0.0)
            part = _dg(oh, pay, (((0,), (0,)), ((), ())))
            idx = pl.ds(sc * 128, 128)
            sorted_ref[idx, :] = sorted_ref[idx, :] + part
            return c2

        lax.fori_loop(0, _NBLK, scat_s, 0)
        return carry

    lax.fori_loop(0, _NROWS, scat_i, 0)

    surv_ref[...] = jnp.zeros((_SCAP, 4), _F32)
    sarea_ref[...] = jnp.zeros((_SCAP, 1), _F32)
    sub512 = lax.broadcasted_iota(jnp.int32, (_SCAP, 1), 0).astype(_F32)
    lane512 = lax.broadcasted_iota(jnp.int32, (128, _SCAP), 1).astype(_F32)

    def blk(b, count):
        gate = (count < float(_NUM_POST_NMS)).astype(_F32)
        blkb = sorted_ref[pl.ds(b * 128, 128), :]
        y1c = blkb[:, 0:1]
        x1c = blkb[:, 1:2]
        y2c = blkb[:, 2:3]
        x2c = blkb[:, 3:4]
        areac = (jnp.maximum(y2c - y1c, 0.0)
                 * jnp.maximum(x2c - x1c, 0.0))
        y1r, x1r = trow(y1c), trow(x1c)
        y2r, x2r = trow(y2c), trow(x2c)
        arear = trow(areac)

        sy1 = surv_ref[:, 0:1]
        sx1 = surv_ref[:, 1:2]
        sy2 = surv_ref[:, 2:3]
        sx2 = surv_ref[:, 3:4]
        sa = sarea_ref[...]
        yy1 = jnp.maximum(sy1, y1r)
        xx1 = jnp.maximum(sx1, x1r)
        yy2 = jnp.minimum(sy2, y2r)
        xx2 = jnp.minimum(sx2, x2r)
        inter = (jnp.maximum(yy2 - yy1, 0.0)
                 * jnp.maximum(xx2 - xx1, 0.0))
        iou = inter / (sa + arear - inter + 1e-9)
        valids = sub512 < count
        presup = jnp.max(
            jnp.where((iou > _IOU_THR) & valids, 1.0, 0.0),
            axis=0, keepdims=True)
        invalid = ((b * 128 + lane_i) >= _NUM_PRE_NMS).astype(_F32)
        pre = jnp.maximum(presup, invalid)

        byy1 = jnp.maximum(y1c, y1r)
        bxx1 = jnp.maximum(x1c, x1r)
        byy2 = jnp.minimum(y2c, y2r)
        bxx2 = jnp.minimum(x2c, x2r)
        binter = (jnp.maximum(byy2 - byy1, 0.0)
                  * jnp.maximum(bxx2 - bxx1, 0.0))
        biou = binter / (areac + arear - binter + 1e-9)
        m = jnp.where(biou > _IOU_THR, 1.0, 0.0) * ltmat

        def wcond(c):
            return c[1]

        def wbody(c):
            s, _ = c
            contrib = _dg(1.0 - s, m, (((1,), (0,)), ((), ())))
            s_new = jnp.maximum(pre, (contrib > 0.0).astype(_F32))
            return (s_new, jnp.any(s_new != s))

        s_fin, _ = lax.while_loop(wcond, wbody, (pre, jnp.bool_(True)))

        sel = (1.0 - s_fin) * gate
        epref = _dg(sel, ltmat, (((1,), (0,)), ((), ())))
        pos = count + epref
        posc = tcol(pos)
        selc = tcol(sel)
        oh = jnp.where((posc == lane512) & (selc > 0.0), 1.0, 0.0)
        surv_ref[...] = surv_ref[...] + _dg(
            oh, blkb, (((0,), (0,)), ((), ())))
        sarea_ref[...] = sarea_ref[...] + _dg(
            oh, areac, (((0,), (0,)), ((), ())))
        return count + jnp.sum(sel)

    lax.fori_loop(0, _NBLK, blk, jnp.float32(0.0))
    out_ref[...] = surv_ref[0:_NUM_POST_NMS, :]


def _decode_and_scores(bboxes_txtytwth, anchors, scores, image_shape):
    deltas = bboxes_txtytwth * _STDS + _MEANS
    h = anchors[:, 2] - anchors[:, 0]
    w = anchors[:, 3] - anchors[:, 1]
    cy = anchors[:, 0] + 0.5 * h
    cx = anchors[:, 1] + 0.5 * w
    ncy = cy + deltas[:, 0] * h
    ncx = cx + deltas[:, 1] * w
    nh = h * jnp.exp(deltas[:, 2])
    nw = w * jnp.exp(deltas[:, 3])
    decoded = jnp.stack([ncy - 0.5 * nh, ncx - 0.5 * nw,
                         ncy + 0.5 * nh, ncx + 0.5 * nw], axis=1)
    max_h = image_shape[0].astype(jnp.float32)
    max_w = image_shape[1].astype(jnp.float32)
    y1 = jnp.clip(decoded[:, 0], 0.0, max_h)
    x1 = jnp.clip(decoded[:, 1], 0.0, max_w)
    y2 = jnp.clip(decoded[:, 2], 0.0, max_h)
    x2 = jnp.clip(decoded[:, 3], 0.0, max_w)
    boxes = jnp.stack([y1, x1, y2, x2], axis=1)

    s = jnp.transpose(jnp.reshape(scores, (-1, 2, _NUM_ANCHORS)), (0, 2, 1))
    s = jnp.reshape(s, (-1, 2))
    s = jax.nn.softmax(s, axis=-1)
    s = jnp.transpose(jnp.reshape(s, (-1, _NUM_ANCHORS, 2)), (0, 2, 1))
    s = jnp.reshape(s, (-1, 2 * _NUM_ANCHORS))
    fg = jnp.reshape(s[..., _NUM_ANCHORS:], (-1,))
    return boxes, fg


def kernel(bboxes_txtytwth, anchors, scores, image_shape):
    boxes, fg = _decode_and_scores(bboxes_txtytwth, anchors, scores,
                                   image_shape)
    fgp = jnp.pad(fg, (0, _NP - _N), constant_values=-1.0)
    fg_row = fgp.reshape(_NROWS, 128)
    fg_col = fgp.reshape(_NP, 1)
    boxes_p = jnp.pad(boxes, ((0, _NP - _N), (0, 0)))

    out = pl.pallas_call(
        _nms_body,
        out_shape=jax.ShapeDtypeStruct((_NUM_POST_NMS, 4), jnp.float32),
        scratch_shapes=[
            pltpu.VMEM((_NP, 1), jnp.float32),
            pltpu.VMEM((_KP, 4), jnp.float32),
            pltpu.VMEM((_SCAP, 4), jnp.float32),
            pltpu.VMEM((_SCAP, 1), jnp.float32),
        ],
    )(fg_row, fg_col, boxes_p)
    return out
